# Initial kernel scaffold; baseline (speedup 1.0000x reference)
import jax
import jax.numpy as jnp
from jax.experimental import pallas as pl


def kernel(x, edge_index, W1, b1, W2, b2, Wr, br):
    raise NotImplementedError("write your pallas kernel here")


# speedup vs baseline: 13.0323x; 13.0323x over previous
import functools

import jax
import jax.numpy as jnp
from jax import lax
from jax.experimental import pallas as pl
from jax.experimental.pallas import tpu as pltpu
from jax.experimental.pallas import tpu_sc as plsc

NC = 2
NS = 16
NW = NC * NS
CH = 80


def _sc_mesh():
    return plsc.VectorSubcoreMesh(core_axis_name="c", subcore_axis_name="s")


def _deg_partials(dst, np_pad):
    e = dst.shape[0]
    epw = e // NW
    nchunks = epw // CH
    per_tile = np_pad // NS

    @functools.partial(
        pl.kernel,
        out_type=jax.ShapeDtypeStruct((NC, np_pad), jnp.float32),
        mesh=_sc_mesh(),
        scratch_types=[
            pltpu.VMEM((CH,), jnp.int32),
            pltpu.VMEM((CH,), jnp.float32),
            pltpu.VMEM((per_tile,), jnp.float32),
            pltpu.VMEM_SHARED((np_pad,), jnp.float32),
        ],
    )
    def body(dst_hbm, out_hbm, idx_v, ones_v, stage_v, acc_sh):
        c = lax.axis_index("c")
        s = lax.axis_index("s")

        def fill_ones(i, _):
            ones_v[pl.ds(i * 16, 16)] = jnp.full((16,), 1.0, jnp.float32)
            return 0

        lax.fori_loop(0, CH // 16, fill_ones, 0)

        def fill_zero(i, _):
            stage_v[pl.ds(i * 16, 16)] = jnp.zeros((16,), jnp.float32)
            return 0

        lax.fori_loop(0, per_tile // 16, fill_zero, 0)

        r0 = s * per_tile
        pltpu.sync_copy(stage_v, acc_sh.at[pl.ds(r0, per_tile)])
        plsc.subcore_barrier()

        base = (c * NS + s) * epw

        def step(i, _):
            off = pl.multiple_of(base + i * CH, 8)
            pltpu.sync_copy(dst_hbm.at[pl.ds(off, CH)], idx_v)
            pltpu.sync_copy(ones_v, acc_sh.at[idx_v], add=True)
            return 0

        lax.fori_loop(0, nchunks, step, 0)
        plsc.subcore_barrier()
        pltpu.sync_copy(acc_sh.at[pl.ds(r0, per_tile)],
                        out_hbm.at[c, pl.ds(r0, per_tile)])

    return body(dst)


def _message_partials(hp, src, dst, zrows):
    np_pad, d = hp.shape
    e = src.shape[0]
    epw = e // NW
    nchunks = epw // CH
    per_tile = np_pad // NS

    @functools.partial(
        pl.kernel,
        out_type=(jax.ShapeDtypeStruct((np_pad, d), jnp.float32),
                  jax.ShapeDtypeStruct((np_pad, d), jnp.float32)),
        mesh=_sc_mesh(),
        scratch_types=[
            pltpu.VMEM((CH,), jnp.int32),
            pltpu.VMEM((CH,), jnp.int32),
            pltpu.VMEM((CH, d), jnp.float32),
            pltpu.VMEM_SHARED((np_pad, d), jnp.float32),
            pltpu.SemaphoreType.DMA,
        ],
    )
    def body(hp_hbm, src_hbm, dst_hbm, zr_hbm, p0_hbm, p1_hbm,
             srcv, dstv, rows_v, acc_sh, sem):
        c = lax.axis_index("c")
        s = lax.axis_index("s")
        r0 = s * per_tile

        @pl.when(c == 0)
        def _():
            pltpu.sync_copy(hp_hbm.at[pl.ds(r0, per_tile)],
                            acc_sh.at[pl.ds(r0, per_tile)])

        @pl.when(c != 0)
        def _():
            pltpu.sync_copy(zr_hbm, acc_sh.at[pl.ds(r0, per_tile)])

        plsc.subcore_barrier()

        base = (c * NS + s) * epw

        def step(i, _):
            off = pl.multiple_of(base + i * CH, 8)
            pltpu.sync_copy(src_hbm.at[pl.ds(off, CH)], srcv)
            pltpu.sync_copy(dst_hbm.at[pl.ds(off, CH)], dstv)
            pltpu.async_copy(hp_hbm.at[srcv], rows_v, sem).wait()
            pltpu.sync_copy(rows_v, acc_sh.at[dstv], add=True)
            return 0

        lax.fori_loop(0, nchunks, step, 0)
        plsc.subcore_barrier()

        @pl.when(c == 0)
        def _():
            pltpu.sync_copy(acc_sh.at[pl.ds(r0, per_tile)],
                            p0_hbm.at[pl.ds(r0, per_tile)])

        @pl.when(c != 0)
        def _():
            pltpu.sync_copy(acc_sh.at[pl.ds(r0, per_tile)],
                            p1_hbm.at[pl.ds(r0, per_tile)])

    return body(hp, src, dst, zrows)


def _tc_first(x, w1, d0, d1, blk):
    np_pad, d = x.shape

    def body(x_ref, w_ref, d0_ref, d1_ref, hp_ref, dinv_ref):
        dinv = lax.rsqrt(d0_ref[...] + d1_ref[...] + 1.0)
        h = jnp.dot(x_ref[...], w_ref[...],
                    preferred_element_type=jnp.float32)
        hp_ref[...] = h * dinv
        dinv_ref[...] = dinv

    return pl.pallas_call(
        body,
        grid=(np_pad // blk,),
        in_specs=[
            pl.BlockSpec((blk, d), lambda i: (i, 0)),
            pl.BlockSpec((d, d), lambda i: (0, 0)),
            pl.BlockSpec((blk, 1), lambda i: (i, 0)),
            pl.BlockSpec((blk, 1), lambda i: (i, 0)),
        ],
        out_specs=[
            pl.BlockSpec((blk, d), lambda i: (i, 0)),
            pl.BlockSpec((blk, 1), lambda i: (i, 0)),
        ],
        out_shape=[
            jax.ShapeDtypeStruct((np_pad, d), jnp.float32),
            jax.ShapeDtypeStruct((np_pad, 1), jnp.float32),
        ],
    )(x, w1, d0, d1)


def _tc_mid(p0, p1, dinv, b, w2, blk):
    np_pad, d = p0.shape

    def body(p0_ref, p1_ref, dinv_ref, b_ref, w_ref, hp_ref):
        h = jnp.maximum((p0_ref[...] + p1_ref[...]) * dinv_ref[...]
                        + b_ref[...], 0.0)
        hp_ref[...] = jnp.dot(h, w_ref[...],
                              preferred_element_type=jnp.float32) * dinv_ref[...]

    return pl.pallas_call(
        body,
        grid=(np_pad // blk,),
        in_specs=[
            pl.BlockSpec((blk, d), lambda i: (i, 0)),
            pl.BlockSpec((blk, d), lambda i: (i, 0)),
            pl.BlockSpec((blk, 1), lambda i: (i, 0)),
            pl.BlockSpec((1, d), lambda i: (0, 0)),
            pl.BlockSpec((d, d), lambda i: (0, 0)),
        ],
        out_specs=pl.BlockSpec((blk, d), lambda i: (i, 0)),
        out_shape=jax.ShapeDtypeStruct((np_pad, d), jnp.float32),
    )(p0, p1, dinv, b, w2)


def _tc_last(q0, q1, dinv, b, wr, br, blk):
    np_pad, d = q0.shape

    def body(q0_ref, q1_ref, dinv_ref, b_ref, w_ref, br_ref, o_ref):
        h = jnp.maximum((q0_ref[...] + q1_ref[...]) * dinv_ref[...]
                        + b_ref[...], 0.0)
        o_ref[...] = jnp.dot(h, w_ref[...],
                             preferred_element_type=jnp.float32) + br_ref[...]

    return pl.pallas_call(
        body,
        grid=(np_pad // blk,),
        in_specs=[
            pl.BlockSpec((blk, d), lambda i: (i, 0)),
            pl.BlockSpec((blk, d), lambda i: (i, 0)),
            pl.BlockSpec((blk, 1), lambda i: (i, 0)),
            pl.BlockSpec((1, d), lambda i: (0, 0)),
            pl.BlockSpec((d, 1), lambda i: (0, 0)),
            pl.BlockSpec((1, 1), lambda i: (0, 0)),
        ],
        out_specs=pl.BlockSpec((blk, 1), lambda i: (i, 0)),
        out_shape=jax.ShapeDtypeStruct((np_pad, 1), jnp.float32),
    )(q0, q1, dinv, b, wr, br)


def kernel(x, edge_index, W1, b1, W2, b2, Wr, br):
    n, d = x.shape
    np_pad = 10240
    blk = 1024

    src = edge_index[0]
    dst = edge_index[1]
    x_pad = jnp.pad(x, ((0, np_pad - n), (0, 0)))
    zrows = jnp.zeros((np_pad // NS, d), jnp.float32)

    degs = _deg_partials(dst, np_pad)
    d0 = degs[0].reshape(np_pad, 1)
    d1 = degs[1].reshape(np_pad, 1)

    h1p, dinv = _tc_first(x_pad, W1, d0, d1, blk)
    p0, p1 = _message_partials(h1p, src, dst, zrows)
    h2p = _tc_mid(p0, p1, dinv, b1.reshape(1, d), W2, blk)
    q0, q1 = _message_partials(h2p, src, dst, zrows)
    out = _tc_last(q0, q1, dinv, b2.reshape(1, d), Wr, br.reshape(1, 1), blk)
    return out[:n]


# speedup vs baseline: 19.4240x; 1.4905x over previous
import functools

import jax
import jax.numpy as jnp
from jax import lax
from jax.experimental import pallas as pl
from jax.experimental.pallas import tpu as pltpu
from jax.experimental.pallas import tpu_sc as plsc

NC = 2
NS = 16
NW = NC * NS
CH = 80
NPAD = 10240


def _sc_mesh():
    return plsc.VectorSubcoreMesh(core_axis_name="c", subcore_axis_name="s")


def _deg_partials(dst, np_pad):
    e = dst.shape[0]
    epw = e // NW
    nchunks = epw // CH
    per_tile = np_pad // NS

    @functools.partial(
        pl.kernel,
        out_type=jax.ShapeDtypeStruct((NC, np_pad), jnp.float32),
        mesh=_sc_mesh(),
        scratch_types=[
            pltpu.VMEM((CH,), jnp.int32),
            pltpu.VMEM((CH,), jnp.float32),
            pltpu.VMEM((per_tile,), jnp.float32),
            pltpu.VMEM_SHARED((np_pad,), jnp.float32),
        ],
    )
    def body(dst_hbm, out_hbm, idx_v, ones_v, stage_v, acc_sh):
        c = lax.axis_index("c")
        s = lax.axis_index("s")

        def fill_ones(i, _):
            ones_v[pl.ds(i * 16, 16)] = jnp.full((16,), 1.0, jnp.float32)
            return 0

        lax.fori_loop(0, CH // 16, fill_ones, 0)

        def fill_zero(i, _):
            stage_v[pl.ds(i * 16, 16)] = jnp.zeros((16,), jnp.float32)
            return 0

        lax.fori_loop(0, per_tile // 16, fill_zero, 0)

        r0 = s * per_tile
        pltpu.sync_copy(stage_v, acc_sh.at[pl.ds(r0, per_tile)])
        plsc.subcore_barrier()

        base = (c * NS + s) * epw

        def step(i, _):
            off = pl.multiple_of(base + i * CH, 8)
            pltpu.sync_copy(dst_hbm.at[pl.ds(off, CH)], idx_v)
            pltpu.sync_copy(ones_v, acc_sh.at[idx_v], add=True)
            return 0

        lax.fori_loop(0, nchunks, step, 0)
        plsc.subcore_barrier()
        pltpu.sync_copy(acc_sh.at[pl.ds(r0, per_tile)],
                        out_hbm.at[c, pl.ds(r0, per_tile)])

    return body(dst)


def _message_partials(hp, src, dst, zrows):
    np_pad, d = hp.shape
    e = src.shape[0]
    epw = e // NW
    nchunks = epw // CH
    per_tile = np_pad // NS
    half = (nchunks - 1) // 2

    @functools.partial(
        pl.kernel,
        out_type=(jax.ShapeDtypeStruct((np_pad, d), jnp.float32),
                  jax.ShapeDtypeStruct((np_pad, d), jnp.float32)),
        mesh=_sc_mesh(),
        scratch_types=[
            pltpu.VMEM((CH,), jnp.int32),
            pltpu.VMEM((CH,), jnp.int32),
            pltpu.VMEM((CH,), jnp.int32),
            pltpu.VMEM((CH,), jnp.int32),
            pltpu.VMEM((CH, d), jnp.float32),
            pltpu.VMEM((CH, d), jnp.float32),
            pltpu.VMEM_SHARED((np_pad, d), jnp.float32),
            pltpu.SemaphoreType.DMA,
            pltpu.SemaphoreType.DMA,
        ],
    )
    def body(hp_hbm, src_hbm, dst_hbm, zr_hbm, p0_hbm, p1_hbm,
             src0, src1, dst0, dst1, rows0, rows1, acc_sh, g0, g1):
        srcB = [src0, src1]
        dstB = [dst0, dst1]
        rows = [rows0, rows1]
        gsem = [g0, g1]
        c = lax.axis_index("c")
        s = lax.axis_index("s")
        r0 = s * per_tile
        base = (c * NS + s) * epw

        @pl.when(c == 0)
        def _():
            pltpu.sync_copy(hp_hbm.at[pl.ds(r0, per_tile)],
                            acc_sh.at[pl.ds(r0, per_tile)])

        @pl.when(c != 0)
        def _():
            pltpu.sync_copy(zr_hbm, acc_sh.at[pl.ds(r0, per_tile)])

        plsc.subcore_barrier()

        def load_idx(i, b):
            off = pl.multiple_of(base + i * CH, 8)
            pltpu.sync_copy(src_hbm.at[pl.ds(off, CH)], srcB[b])
            pltpu.sync_copy(dst_hbm.at[pl.ds(off, CH)], dstB[b])

        def issue_gather(b):
            pltpu.async_copy(hp_hbm.at[srcB[b]], rows[b], gsem[b])

        def wait_gather(b):
            pltpu.make_async_copy(hp_hbm.at[srcB[b]], rows[b],
                                  gsem[b]).wait()

        def scatter(b):
            pltpu.sync_copy(rows[b], acc_sh.at[dstB[b]], add=True)

        load_idx(0, 0)
        issue_gather(0)

        def outer(t, _):
            for b in range(2):
                i = 2 * t + b
                load_idx(i + 1, 1 - b)
                issue_gather(1 - b)
                wait_gather(b)
                scatter(b)
            return 0

        lax.fori_loop(0, half, outer, 0)

        for q in range(2 * half, nchunks):
            b = q % 2
            if q + 1 < nchunks:
                load_idx(q + 1, 1 - b)
                issue_gather(1 - b)
            wait_gather(b)
            scatter(b)

        plsc.subcore_barrier()

        @pl.when(c == 0)
        def _():
            pltpu.sync_copy(acc_sh.at[pl.ds(r0, per_tile)],
                            p0_hbm.at[pl.ds(r0, per_tile)])

        @pl.when(c != 0)
        def _():
            pltpu.sync_copy(acc_sh.at[pl.ds(r0, per_tile)],
                            p1_hbm.at[pl.ds(r0, per_tile)])

    return body(hp, src, dst, zrows)


def _tc_first(x, w1, d0, d1, blk):
    np_pad, d = x.shape

    def body(x_ref, w_ref, d0_ref, d1_ref, hp_ref, dinv_ref):
        dinv = lax.rsqrt(d0_ref[...] + d1_ref[...] + 1.0)
        h = jnp.dot(x_ref[...], w_ref[...],
                    preferred_element_type=jnp.float32)
        hp_ref[...] = h * dinv
        dinv_ref[...] = dinv

    return pl.pallas_call(
        body,
        grid=(np_pad // blk,),
        in_specs=[
            pl.BlockSpec((blk, d), lambda i: (i, 0)),
            pl.BlockSpec((d, d), lambda i: (0, 0)),
            pl.BlockSpec((blk, 1), lambda i: (i, 0)),
            pl.BlockSpec((blk, 1), lambda i: (i, 0)),
        ],
        out_specs=[
            pl.BlockSpec((blk, d), lambda i: (i, 0)),
            pl.BlockSpec((blk, 1), lambda i: (i, 0)),
        ],
        out_shape=[
            jax.ShapeDtypeStruct((np_pad, d), jnp.float32),
            jax.ShapeDtypeStruct((np_pad, 1), jnp.float32),
        ],
    )(x, w1, d0, d1)


def _tc_mid(p0, p1, dinv, b, w2, blk):
    np_pad, d = p0.shape

    def body(p0_ref, p1_ref, dinv_ref, b_ref, w_ref, hp_ref):
        h = jnp.maximum((p0_ref[...] + p1_ref[...]) * dinv_ref[...]
                        + b_ref[...], 0.0)
        hp_ref[...] = jnp.dot(h, w_ref[...],
                              preferred_element_type=jnp.float32) * dinv_ref[...]

    return pl.pallas_call(
        body,
        grid=(np_pad // blk,),
        in_specs=[
            pl.BlockSpec((blk, d), lambda i: (i, 0)),
            pl.BlockSpec((blk, d), lambda i: (i, 0)),
            pl.BlockSpec((blk, 1), lambda i: (i, 0)),
            pl.BlockSpec((1, d), lambda i: (0, 0)),
            pl.BlockSpec((d, d), lambda i: (0, 0)),
        ],
        out_specs=pl.BlockSpec((blk, d), lambda i: (i, 0)),
        out_shape=jax.ShapeDtypeStruct((np_pad, d), jnp.float32),
    )(p0, p1, dinv, b, w2)


def _tc_last(q0, q1, dinv, b, wr, br, blk):
    np_pad, d = q0.shape

    def body(q0_ref, q1_ref, dinv_ref, b_ref, w_ref, br_ref, o_ref):
        h = jnp.maximum((q0_ref[...] + q1_ref[...]) * dinv_ref[...]
                        + b_ref[...], 0.0)
        o_ref[...] = jnp.dot(h, w_ref[...],
                             preferred_element_type=jnp.float32) + br_ref[...]

    return pl.pallas_call(
        body,
        grid=(np_pad // blk,),
        in_specs=[
            pl.BlockSpec((blk, d), lambda i: (i, 0)),
            pl.BlockSpec((blk, d), lambda i: (i, 0)),
            pl.BlockSpec((blk, 1), lambda i: (i, 0)),
            pl.BlockSpec((1, d), lambda i: (0, 0)),
            pl.BlockSpec((d, 1), lambda i: (0, 0)),
            pl.BlockSpec((1, 1), lambda i: (0, 0)),
        ],
        out_specs=pl.BlockSpec((blk, 1), lambda i: (i, 0)),
        out_shape=jax.ShapeDtypeStruct((np_pad, 1), jnp.float32),
    )(q0, q1, dinv, b, wr, br)


def kernel(x, edge_index, W1, b1, W2, b2, Wr, br):
    n, d = x.shape
    blk = 1024

    src = edge_index[0]
    dst = edge_index[1]
    x_pad = jnp.pad(x, ((0, NPAD - n), (0, 0)))
    zrows = jnp.zeros((NPAD // NS, d), jnp.float32)

    degs = _deg_partials(dst, NPAD)
    d0 = degs[0].reshape(NPAD, 1)
    d1 = degs[1].reshape(NPAD, 1)

    h1p, dinv = _tc_first(x_pad, W1, d0, d1, blk)
    p0, p1 = _message_partials(h1p, src, dst, zrows)
    h2p = _tc_mid(p0, p1, dinv, b1.reshape(1, d), W2, blk)
    q0, q1 = _message_partials(h2p, src, dst, zrows)
    out = _tc_last(q0, q1, dinv, b2.reshape(1, d), Wr, br.reshape(1, 1), blk)
    return out[:n]
